# baseline (speedup 1.0000x reference)
import jax
import jax.numpy as jnp
from jax.experimental import pallas as pl
from jax.experimental.pallas import tpu as pltpu

_NSLICES = 4


def _dma_copy_body(in_ref, out_ref, sems):
    rows = in_ref.shape[0]
    chunk = rows // _NSLICES
    for i in range(_NSLICES):
        pltpu.make_async_copy(
            in_ref.at[pl.ds(i * chunk, chunk), :],
            out_ref.at[pl.ds(i * chunk, chunk), :],
            sems.at[i],
        ).start()
    for i in range(_NSLICES):
        pltpu.make_async_copy(
            in_ref.at[pl.ds(i * chunk, chunk), :],
            out_ref.at[pl.ds(i * chunk, chunk), :],
            sems.at[i],
        ).wait()


def kernel(queue):
    return pl.pallas_call(
        _dma_copy_body,
        out_shape=jax.ShapeDtypeStruct(queue.shape, queue.dtype),
        in_specs=[pl.BlockSpec(memory_space=pl.ANY)],
        out_specs=pl.BlockSpec(memory_space=pl.ANY),
        scratch_shapes=[pltpu.SemaphoreType.DMA((_NSLICES,))],
    )(queue)


# speedup vs baseline: 48.2875x; 48.2875x over previous
import jax
import jax.numpy as jnp
from jax.experimental import pallas as pl
from jax.experimental.pallas import tpu as pltpu

_BLK = 8192


def _copy_body(in_ref, out_ref):
    out_ref[...] = in_ref[...]


def kernel(queue):
    rows, dim = queue.shape
    return pl.pallas_call(
        _copy_body,
        out_shape=jax.ShapeDtypeStruct(queue.shape, queue.dtype),
        grid=(rows // _BLK,),
        in_specs=[pl.BlockSpec((_BLK, dim), lambda i: (i, 0))],
        out_specs=pl.BlockSpec((_BLK, dim), lambda i: (i, 0)),
        compiler_params=pltpu.CompilerParams(
            dimension_semantics=("parallel",),
        ),
    )(queue)


# speedup vs baseline: 49.1196x; 1.0172x over previous
import jax
import jax.numpy as jnp
from jax.experimental import pallas as pl
from jax.experimental.pallas import tpu as pltpu

_BLK = 16384


def _copy_body(in_ref, out_ref):
    out_ref[...] = in_ref[...]


def kernel(queue):
    rows, dim = queue.shape
    return pl.pallas_call(
        _copy_body,
        out_shape=jax.ShapeDtypeStruct(queue.shape, queue.dtype),
        grid=(rows // _BLK,),
        in_specs=[pl.BlockSpec((_BLK, dim), lambda i: (i, 0))],
        out_specs=pl.BlockSpec((_BLK, dim), lambda i: (i, 0)),
        compiler_params=pltpu.CompilerParams(
            dimension_semantics=("parallel",),
        ),
    )(queue)
